# baseline (speedup 1.0000x reference)
import jax
import jax.numpy as jnp
from jax import lax
from jax.experimental import pallas as pl
from jax.experimental.pallas import tpu as pltpu
from jax.experimental.pallas import tpu_sc as plsc

N_POINTS = 320000
D = 128
N_VOX = 10000

NC = 2
NS = 16
NW = NC * NS
PTS_PER_TILE = N_POINTS // NW
CHUNK = 80
CHUNKS_PER_TILE = PTS_PER_TILE // CHUNK
GCH = 80
NG = PTS_PER_TILE // GCH
N_VOX_PAD = 10240
VOX_PER_TILE = N_VOX_PAD // NS
ZROWS = 80


def _make_sc_scatter():
    mesh = plsc.VectorSubcoreMesh(core_axis_name="c", subcore_axis_name="s",
                                  num_cores=NC, num_subcores=NS)

    def sc_entry(feat_hbm, coors_hbm, zeros_hbm, zeros1_hbm, ones_hbm,
                 sums_hbm, cnts_hbm, idx_v, st_a, st_b, st_c, ones_v, cb_v,
                 sums_sh, cnts_sh, gs_a, gs_b, gs_c, ss, cs, fs):
        c = lax.axis_index("c")
        s = lax.axis_index("s")
        wid = s * NC + c

        idesc = pltpu.async_copy(coors_hbm.at[wid], idx_v, gs_b)
        odesc = pltpu.async_copy(ones_hbm, ones_v, gs_c)

        pltpu.sync_copy(zeros_hbm, st_a.at[pl.ds(0, ZROWS)])
        zdescs = []
        for q in range(VOX_PER_TILE // ZROWS):
            zdescs.append(pltpu.async_copy(
                st_a.at[pl.ds(0, ZROWS)],
                sums_sh.at[pl.ds(s * VOX_PER_TILE + q * ZROWS, ZROWS)], fs))
        pltpu.sync_copy(zeros1_hbm, cb_v)
        pltpu.sync_copy(cb_v, cnts_sh.at[pl.ds(s * VOX_PER_TILE, VOX_PER_TILE)])
        for d in zdescs:
            d.wait()
        idesc.wait()
        odesc.wait()
        plsc.subcore_barrier()

        ring = ((st_a, gs_a), (st_b, gs_b), (st_c, gs_c))

        def gather(j, buf, sem):
            base = wid * PTS_PER_TILE + j * GCH
            pltpu.async_copy(feat_hbm.at[pl.ds(base, GCH)], buf, sem)

        def gwait(buf, sem):
            pltpu.make_async_copy(feat_hbm.at[pl.ds(0, GCH)], buf, sem).wait()

        def swait():
            pltpu.make_async_copy(st_a, sums_sh.at[idx_v.at[0]], ss).wait()
            pltpu.make_async_copy(ones_v, cnts_sh.at[idx_v.at[0]], cs).wait()

        def substep(j, slot):
            buf, sem = ring[slot]
            gwait(buf, sem)

            @pl.when(j + 1 <= NG - 1)
            def _():
                gather(j + 1, *ring[(slot + 1) % 3])

            pltpu.async_copy(buf, sums_sh.at[idx_v.at[j]], ss, add=True)
            pltpu.async_copy(ones_v, cnts_sh.at[idx_v.at[j]], cs, add=True)

            @pl.when(j >= 1)
            def _():
                swait()

        gather(0, st_a, gs_a)

        def body(k, carry):
            j0 = 3 * k
            substep(j0, 0)
            substep(j0 + 1, 1)
            substep(j0 + 2, 2)
            return carry

        lax.fori_loop(0, NG // 3, body, 0)
        substep(NG - 2, (NG - 2) % 3)
        substep(NG - 1, (NG - 1) % 3)
        swait()

        plsc.subcore_barrier()

        descs = []
        for q in range(VOX_PER_TILE // ZROWS):
            off = s * VOX_PER_TILE + q * ZROWS
            buf = st_a if q % 2 == 0 else st_b
            if q >= 2:
                descs[q - 2].wait()
            pltpu.sync_copy(sums_sh.at[pl.ds(off, ZROWS)],
                            buf.at[pl.ds(0, ZROWS)])
            descs.append(pltpu.async_copy(buf.at[pl.ds(0, ZROWS)],
                                          sums_hbm.at[c, pl.ds(off, ZROWS)],
                                          fs))
        pltpu.sync_copy(cnts_sh.at[pl.ds(s * VOX_PER_TILE, VOX_PER_TILE)], cb_v)
        pltpu.sync_copy(cb_v, cnts_hbm.at[c, s])
        descs[-2].wait()
        descs[-1].wait()

    return pl.kernel(
        sc_entry,
        out_type=[
            jax.ShapeDtypeStruct((NC, N_VOX_PAD, D), jnp.float32),
            jax.ShapeDtypeStruct((NC, NS, VOX_PER_TILE), jnp.float32),
        ],
        mesh=mesh,
        scratch_types=[
            pltpu.VMEM((CHUNKS_PER_TILE, CHUNK), jnp.int32),
            pltpu.VMEM((GCH, D), jnp.float32),
            pltpu.VMEM((GCH, D), jnp.float32),
            pltpu.VMEM((GCH, D), jnp.float32),
            pltpu.VMEM((CHUNK,), jnp.float32),
            pltpu.VMEM((VOX_PER_TILE,), jnp.float32),
            pltpu.VMEM_SHARED((N_VOX_PAD, D), jnp.float32),
            pltpu.VMEM_SHARED((N_VOX_PAD,), jnp.float32),
            pltpu.SemaphoreType.DMA,
            pltpu.SemaphoreType.DMA,
            pltpu.SemaphoreType.DMA,
            pltpu.SemaphoreType.DMA,
            pltpu.SemaphoreType.DMA,
            pltpu.SemaphoreType.DMA,
        ],
    )


_sc_scatter = _make_sc_scatter()

_ROWS_BLK = 1000


def _combine_body(sums_ref, cnts_ref, out_ref):
    s = sums_ref[0] + sums_ref[1]
    cnt = cnts_ref[0] + cnts_ref[1]
    out_ref[...] = s / jnp.maximum(cnt, 1.0)


_combine = pl.pallas_call(
    _combine_body,
    grid=(N_VOX // _ROWS_BLK,),
    in_specs=[
        pl.BlockSpec((NC, _ROWS_BLK, D), lambda i: (0, i, 0)),
        pl.BlockSpec((NC, _ROWS_BLK, 1), lambda i: (0, i, 0)),
    ],
    out_specs=pl.BlockSpec((_ROWS_BLK, D), lambda i: (i, 0)),
    out_shape=jax.ShapeDtypeStruct((N_VOX, D), jnp.float32),
)


def _probe_body(x_ref, o_ref):
    o_ref[...] = x_ref[...] * 2.0


_probe = pl.pallas_call(
    _probe_body,
    grid=(32,),
    in_specs=[pl.BlockSpec((1024, D), lambda i: (i, 0))],
    out_specs=pl.BlockSpec((1024, D), lambda i: (i, 0)),
    out_shape=jax.ShapeDtypeStruct((32768, D), jnp.float32),
)


@jax.jit
def kernel(features, coors):
    coors2d = coors.reshape(NW, CHUNKS_PER_TILE, CHUNK)
    zeros = jnp.zeros((ZROWS, D), jnp.float32)
    zeros1 = jnp.zeros((VOX_PER_TILE,), jnp.float32)
    ones = jnp.ones((CHUNK,), jnp.float32)
    sums_p, cnts_p = _sc_scatter(features, coors2d, zeros, zeros1, ones)
    cnts_col = cnts_p.reshape(NC, N_VOX_PAD, 1)
    probe_out = _probe(features[:32768])
    voxel_features = _combine(sums_p, cnts_col) + 0.0 * probe_out[:1, :]
    features_coors = jnp.arange(N_VOX, dtype=coors.dtype)
    return voxel_features, features_coors


# speedup vs baseline: 1.0765x; 1.0765x over previous
import jax
import jax.numpy as jnp
from jax import lax
from jax.experimental import pallas as pl
from jax.experimental.pallas import tpu as pltpu
from jax.experimental.pallas import tpu_sc as plsc

N_POINTS = 320000
D = 128
N_VOX = 10000

NC = 2
NS = 16
NW = NC * NS
PTS_PER_TILE = N_POINTS // NW
CHUNK = 80
CHUNKS_PER_TILE = PTS_PER_TILE // CHUNK
GCH = 80
NG = PTS_PER_TILE // GCH
N_VOX_PAD = 10240
VOX_PER_TILE = N_VOX_PAD // NS
ZROWS = 80


def _make_sc_scatter():
    mesh = plsc.VectorSubcoreMesh(core_axis_name="c", subcore_axis_name="s",
                                  num_cores=NC, num_subcores=NS)

    def sc_entry(feat_hbm, coors_hbm, zeros_hbm, zeros1_hbm, ones_hbm,
                 sums_hbm, cnts_hbm, idx_v, st_a, st_b, st_c, ones_v, cb_v,
                 sums_sh, cnts_sh, gs_a, gs_b, gs_c, ss, cs, fs):
        c = lax.axis_index("c")
        s = lax.axis_index("s")
        wid = s * NC + c

        idesc = pltpu.async_copy(coors_hbm.at[wid], idx_v, gs_b)
        odesc = pltpu.async_copy(ones_hbm, ones_v, gs_c)

        pltpu.sync_copy(zeros_hbm, st_a.at[pl.ds(0, ZROWS)])
        zdescs = []
        for q in range(VOX_PER_TILE // ZROWS):
            zdescs.append(pltpu.async_copy(
                st_a.at[pl.ds(0, ZROWS)],
                sums_sh.at[pl.ds(s * VOX_PER_TILE + q * ZROWS, ZROWS)], fs))
        pltpu.sync_copy(zeros1_hbm, cb_v)
        pltpu.sync_copy(cb_v, cnts_sh.at[pl.ds(s * VOX_PER_TILE, VOX_PER_TILE)])
        for d in zdescs:
            d.wait()
        idesc.wait()
        odesc.wait()
        plsc.subcore_barrier()

        ring = ((st_a, gs_a), (st_b, gs_b), (st_c, gs_c))

        def gather(j, buf, sem):
            base = wid * PTS_PER_TILE + j * GCH
            pltpu.async_copy(feat_hbm.at[pl.ds(base, GCH)], buf, sem)

        def gwait(buf, sem):
            pltpu.make_async_copy(feat_hbm.at[pl.ds(0, GCH)], buf, sem).wait()

        def swait():
            pltpu.make_async_copy(st_a, sums_sh.at[idx_v.at[0]], ss).wait()
            pltpu.make_async_copy(ones_v, cnts_sh.at[idx_v.at[0]], cs).wait()

        def substep(j, slot):
            buf, sem = ring[slot]
            gwait(buf, sem)

            @pl.when(j + 1 <= NG - 1)
            def _():
                gather(j + 1, *ring[(slot + 1) % 3])

            pltpu.async_copy(buf, sums_sh.at[idx_v.at[j]], ss, add=True)
            pltpu.async_copy(ones_v, cnts_sh.at[idx_v.at[j]], cs, add=True)

            @pl.when(j >= 1)
            def _():
                swait()

        gather(0, st_a, gs_a)

        def body(k, carry):
            j0 = 3 * k
            substep(j0, 0)
            substep(j0 + 1, 1)
            substep(j0 + 2, 2)
            return carry

        lax.fori_loop(0, NG // 3, body, 0)
        substep(NG - 2, (NG - 2) % 3)
        substep(NG - 1, (NG - 1) % 3)
        swait()

        plsc.subcore_barrier()

        descs = []
        for q in range(VOX_PER_TILE // ZROWS):
            off = s * VOX_PER_TILE + q * ZROWS
            buf = st_a if q % 2 == 0 else st_b
            if q >= 2:
                descs[q - 2].wait()
            pltpu.sync_copy(sums_sh.at[pl.ds(off, ZROWS)],
                            buf.at[pl.ds(0, ZROWS)])
            descs.append(pltpu.async_copy(buf.at[pl.ds(0, ZROWS)],
                                          sums_hbm.at[c, pl.ds(off, ZROWS)],
                                          fs))
        pltpu.sync_copy(cnts_sh.at[pl.ds(s * VOX_PER_TILE, VOX_PER_TILE)], cb_v)
        pltpu.sync_copy(cb_v, cnts_hbm.at[c, s])
        descs[-2].wait()
        descs[-1].wait()

    return pl.kernel(
        sc_entry,
        out_type=[
            jax.ShapeDtypeStruct((NC, N_VOX_PAD, D), jnp.float32),
            jax.ShapeDtypeStruct((NC, NS, VOX_PER_TILE), jnp.float32),
        ],
        mesh=mesh,
        scratch_types=[
            pltpu.VMEM((CHUNKS_PER_TILE, CHUNK), jnp.int32),
            pltpu.VMEM((GCH, D), jnp.float32),
            pltpu.VMEM((GCH, D), jnp.float32),
            pltpu.VMEM((GCH, D), jnp.float32),
            pltpu.VMEM((CHUNK,), jnp.float32),
            pltpu.VMEM((VOX_PER_TILE,), jnp.float32),
            pltpu.VMEM_SHARED((N_VOX_PAD, D), jnp.float32),
            pltpu.VMEM_SHARED((N_VOX_PAD,), jnp.float32),
            pltpu.SemaphoreType.DMA,
            pltpu.SemaphoreType.DMA,
            pltpu.SemaphoreType.DMA,
            pltpu.SemaphoreType.DMA,
            pltpu.SemaphoreType.DMA,
            pltpu.SemaphoreType.DMA,
        ],
    )


_sc_scatter = _make_sc_scatter()

_ROWS_BLK = 1000


def _combine_body(sums_ref, cnts_ref, out_ref):
    s = sums_ref[0] + sums_ref[1]
    cnt = cnts_ref[0] + cnts_ref[1]
    out_ref[...] = s / jnp.maximum(cnt, 1.0)


_combine = pl.pallas_call(
    _combine_body,
    grid=(N_VOX // _ROWS_BLK,),
    in_specs=[
        pl.BlockSpec((NC, _ROWS_BLK, D), lambda i: (0, i, 0)),
        pl.BlockSpec((NC, _ROWS_BLK, 1), lambda i: (0, i, 0)),
    ],
    out_specs=pl.BlockSpec((_ROWS_BLK, D), lambda i: (i, 0)),
    out_shape=jax.ShapeDtypeStruct((N_VOX, D), jnp.float32),
)


@jax.jit
def kernel(features, coors):
    coors2d = coors.reshape(NW, CHUNKS_PER_TILE, CHUNK)
    zeros = jnp.zeros((ZROWS, D), jnp.float32)
    zeros1 = jnp.zeros((VOX_PER_TILE,), jnp.float32)
    ones = jnp.ones((CHUNK,), jnp.float32)
    sums_p, cnts_p = _sc_scatter(features, coors2d, zeros, zeros1, ones)
    cnts_col = cnts_p.reshape(NC, N_VOX_PAD, 1)
    voxel_features = _combine(sums_p, cnts_col)
    features_coors = jnp.arange(N_VOX, dtype=coors.dtype)
    return voxel_features, features_coors
